# Initial kernel scaffold; baseline (speedup 1.0000x reference)
import jax
import jax.numpy as jnp
from jax.experimental import pallas as pl


def kernel(hidden_states, codebooks):
    raise NotImplementedError("write your pallas kernel here")


# speedup vs baseline: 11.3501x; 11.3501x over previous
import jax
import jax.numpy as jnp
from jax.experimental import pallas as pl
from jax.experimental.pallas import tpu as pltpu

DIM = 256
N_Q = 8
BINS = 1024
N_TOK = 2048
TOK_BLOCK = 256

_PREC = jax.lax.Precision.HIGHEST


def _dot(a, b, dims):
    return jax.lax.dot_general(a, b, (dims, ((), ())),
                               precision=_PREC,
                               preferred_element_type=jnp.float32)


def _rvq_body(h_ref, cb_ref, codes_ref, quant_ref, r_ref):
    lvl = pl.program_id(1)

    @pl.when(lvl == 0)
    def _():
        r_ref[...] = h_ref[...]
        quant_ref[...] = jnp.zeros_like(quant_ref)

    r = r_ref[...]
    cb = cb_ref[0]
    lane = jax.lax.broadcasted_iota(jnp.int32, (r.shape[0], BINS), 1)
    ones_row = jnp.ones((1, DIM), jnp.float32)
    cnorm = _dot(ones_row, cb * cb, ((1,), (1,)))
    rc = _dot(r, cb, ((1,), (1,)))
    scores = cnorm - 2.0 * rc
    m1 = jnp.min(scores, axis=1, keepdims=True)
    i1 = jnp.min(jnp.where(scores == m1, lane, BINS), axis=1, keepdims=True)
    masked = jnp.where(lane == i1, jnp.inf, scores)
    m2 = jnp.min(masked, axis=1, keepdims=True)
    i2 = jnp.min(jnp.where(masked == m2, lane, BINS), axis=1, keepdims=True)
    oh1 = (lane == i1).astype(jnp.float32)
    oh2 = (lane == i2).astype(jnp.float32)
    c1 = _dot(oh1, cb, ((1,), (0,)))
    c2 = _dot(oh2, cb, ((1,), (0,)))
    d1 = jnp.sum((r - c1) ** 2, axis=1, keepdims=True)
    d2 = jnp.sum((r - c2) ** 2, axis=1, keepdims=True)
    pick2 = (d2 < d1) | ((d2 == d1) & (i2 < i1))
    chosen = jnp.where(pick2, c2, c1)
    ohc = jnp.where(pick2, oh2, oh1)
    lane_row = jax.lax.broadcasted_iota(
        jnp.int32, (1, BINS), 1).astype(jnp.float32)
    idx_row = _dot(lane_row, ohc, ((1,), (1,)))
    codes_ref[0] = idx_row.astype(jnp.int32)
    quant_ref[...] += chosen
    r_ref[...] = r - chosen


def kernel(hidden_states, codebooks):
    grid = (N_TOK // TOK_BLOCK, N_Q)
    codes3, quantized = pl.pallas_call(
        _rvq_body,
        grid=grid,
        in_specs=[
            pl.BlockSpec((TOK_BLOCK, DIM), lambda j, i: (j, 0)),
            pl.BlockSpec((1, BINS, DIM), lambda j, i: (i, 0, 0)),
        ],
        out_specs=[
            pl.BlockSpec((1, 1, TOK_BLOCK), lambda j, i: (i, 0, j)),
            pl.BlockSpec((TOK_BLOCK, DIM), lambda j, i: (j, 0)),
        ],
        out_shape=[
            jax.ShapeDtypeStruct((N_Q, 1, N_TOK), jnp.int32),
            jax.ShapeDtypeStruct((N_TOK, DIM), jnp.float32),
        ],
        scratch_shapes=[pltpu.VMEM((TOK_BLOCK, DIM), jnp.float32)],
    )(hidden_states, codebooks)
    return codes3.reshape(N_Q, N_TOK), quantized


# speedup vs baseline: 25.5394x; 2.2501x over previous
import jax
import jax.numpy as jnp
from jax.experimental import pallas as pl
from jax.experimental.pallas import tpu as pltpu

DIM = 256
N_Q = 8
BINS = 1024
N_TOK = 2048
TOK_BLOCK = 256


def _dot(a, b, dims):
    return jax.lax.dot_general(a, b, (dims, ((), ())),
                               preferred_element_type=jnp.float32)


def _rvq_body(h_ref, hi_ref, mid_ref, lo_ref, codes_ref, quant_ref,
              r_ref, cn_ref):
    jblk = pl.program_id(0)
    lvl = pl.program_id(1)

    c_hi = hi_ref[0]
    c_mid = mid_ref[0]
    c_lo = lo_ref[0]

    @pl.when(jblk == 0)
    def _():
        cb = (c_hi.astype(jnp.float32) + c_mid.astype(jnp.float32)
              ) + c_lo.astype(jnp.float32)
        cbsq = cb * cb
        sq_hi = cbsq.astype(jnp.bfloat16)
        sq_lo = (cbsq - sq_hi.astype(jnp.float32)).astype(jnp.bfloat16)
        ones_row = jnp.ones((1, DIM), jnp.bfloat16)
        cn_ref[lvl] = (_dot(ones_row, sq_hi, ((1,), (1,)))
                       + _dot(ones_row, sq_lo, ((1,), (1,))))

    @pl.when(lvl == 0)
    def _():
        r_ref[...] = h_ref[...]
        quant_ref[...] = jnp.zeros_like(quant_ref)

    r = r_ref[...]
    r_hi = r.astype(jnp.bfloat16)
    r_lo = (r - r_hi.astype(jnp.float32)).astype(jnp.bfloat16)
    lane = jax.lax.broadcasted_iota(jnp.int32, (r.shape[0], BINS), 1)
    rc = (_dot(r_hi, c_hi, ((1,), (1,)))
          + _dot(r_hi, c_mid, ((1,), (1,)))
          + _dot(r_lo, c_hi, ((1,), (1,))))
    scores = cn_ref[lvl] - 2.0 * rc
    m1 = jnp.min(scores, axis=1, keepdims=True)
    i1 = jnp.min(jnp.where(scores == m1, lane, BINS), axis=1, keepdims=True)
    masked = jnp.where(lane == i1, jnp.inf, scores)
    m2 = jnp.min(masked, axis=1, keepdims=True)
    i2 = jnp.min(jnp.where(masked == m2, lane, BINS), axis=1, keepdims=True)
    oh = jnp.concatenate(
        [(lane == i1).astype(jnp.bfloat16),
         (lane == i2).astype(jnp.bfloat16)], axis=0)
    c12 = (_dot(oh, c_hi, ((1,), (0,)))
           + _dot(oh, c_mid, ((1,), (0,)))
           + _dot(oh, c_lo, ((1,), (0,))))
    t = r.shape[0]
    c1 = c12[:t]
    c2 = c12[t:]
    d1 = jnp.sum((r - c1) ** 2, axis=1, keepdims=True)
    d2 = jnp.sum((r - c2) ** 2, axis=1, keepdims=True)
    pick2 = (d2 < d1) | ((d2 == d1) & (i2 < i1))
    chosen = jnp.where(pick2, c2, c1)
    ohc = jnp.where(pick2, oh[t:], oh[:t])
    li = jax.lax.broadcasted_iota(jnp.int32, (2, BINS), 1)
    ab = jnp.where(jax.lax.broadcasted_iota(jnp.int32, (2, BINS), 0) == 0,
                   li // 256, li % 256).astype(jnp.bfloat16)
    ab_t = _dot(ab, ohc, ((1,), (1,)))
    idx_row = ab_t[:1] * 256.0 + ab_t[1:]
    codes_ref[0] = idx_row.astype(jnp.int32)
    quant_ref[...] += chosen
    r_ref[...] = r - chosen


def kernel(hidden_states, codebooks):
    cb_hi = codebooks.astype(jnp.bfloat16)
    res1 = codebooks - cb_hi.astype(jnp.float32)
    cb_mid = res1.astype(jnp.bfloat16)
    cb_lo = (res1 - cb_mid.astype(jnp.float32)).astype(jnp.bfloat16)

    grid = (N_TOK // TOK_BLOCK, N_Q)
    cb_spec = pl.BlockSpec((1, BINS, DIM), lambda j, i: (i, 0, 0))
    codes3, quantized = pl.pallas_call(
        _rvq_body,
        grid=grid,
        in_specs=[
            pl.BlockSpec((TOK_BLOCK, DIM), lambda j, i: (j, 0)),
            cb_spec, cb_spec, cb_spec,
        ],
        out_specs=[
            pl.BlockSpec((1, 1, TOK_BLOCK), lambda j, i: (i, 0, j)),
            pl.BlockSpec((TOK_BLOCK, DIM), lambda j, i: (j, 0)),
        ],
        out_shape=[
            jax.ShapeDtypeStruct((N_Q, 1, N_TOK), jnp.int32),
            jax.ShapeDtypeStruct((N_TOK, DIM), jnp.float32),
        ],
        scratch_shapes=[
            pltpu.VMEM((TOK_BLOCK, DIM), jnp.float32),
            pltpu.VMEM((N_Q, 1, BINS), jnp.float32),
        ],
    )(hidden_states, cb_hi, cb_mid, cb_lo)
    return codes3.reshape(N_Q, N_TOK), quantized
